# baseline (speedup 1.0000x reference)
import functools

import jax
import jax.numpy as jnp
from jax.experimental import pallas as pl

_BONUS = 10.0


def _wp(v):
    return jnp.concatenate([v[..., 1:], v[..., -1:]], axis=-1)


def _wm(v):
    return jnp.concatenate([v[..., :1], v[..., :-1]], axis=-1)


def _stencil_kernel(ht, n_tiles, xc_ref, xt_ref, xb_ref, coords_ref, y_ref):
    i = pl.program_id(0)
    rs = i * ht
    c = xc_ref[...]
    trow = xt_ref[:, :, pl.ds(jnp.where(i == 0, 0, 7), 1), :]
    brow = xb_ref[:, :, pl.ds(jnp.where(i == n_tiles - 1, 7, 0), 1), :]

    CHp = jnp.concatenate([c[:, :, 1:], brow], axis=2)
    CHm = jnp.concatenate([trow, c[:, :, :-1]], axis=2)
    CWp = _wp(c)
    CWm = _wm(c)

    K = CWm - CWp
    L = CHp - CHm

    LWp = _wp(L)
    LWm = _wm(L)

    m1 = jnp.maximum(jnp.maximum(CHm, c), CHp)
    mh = jnp.maximum(jnp.maximum(_wm(m1), m1), _wp(m1))

    sh = (c.shape[0], c.shape[2], c.shape[3])
    hidx = (rs + jax.lax.broadcasted_iota(jnp.int32, sh, 1)).astype(c.dtype)
    widx = jax.lax.broadcasted_iota(jnp.int32, sh, 2).astype(c.dtype)

    for d in range(4):
        dm = max(d - 1, 0)
        dp = min(d + 1, 3)
        cd = c[:, d]
        two_c = cd + cd
        m00 = 4.0 * ((CWm[:, d] + CWp[:, d]) - two_c)
        m11 = 4.0 * ((CHm[:, d] + CHp[:, d]) - two_c)
        m22 = 4.0 * ((c[:, dm] + c[:, dp]) - two_c)
        m01 = LWp[:, d] - LWm[:, d]
        m12 = L[:, dm] - L[:, dp]
        m02 = K[:, dp] - K[:, dm]
        k0 = K[:, d]
        bu1 = L[:, d]
        bu2 = c[:, dp] - c[:, dm]

        a00 = m11 * m22 - m12 * m12
        a01 = m02 * m12 - m01 * m22
        a02 = m01 * m12 - m02 * m11
        a11 = m00 * m22 - m02 * m02
        a12 = m01 * m02 - m00 * m12
        a22 = m00 * m11 - m01 * m01
        det = m00 * a00 + m01 * a01 + m02 * a02
        valid = det != 0.0
        recip = 2.0 / jnp.where(valid, det, 1.0)

        dot0 = (a01 * bu1 + a02 * bu2) - a00 * k0
        dot1 = (a11 * bu1 + a12 * bu2) - a01 * k0
        dot2 = (a12 * bu1 + a22 * bu2) - a02 * k0

        md = jnp.maximum(jnp.maximum(mh[:, dm], mh[:, d]), mh[:, dp])
        nms = (cd == md) & valid

        amax = jnp.maximum(jnp.maximum(jnp.abs(dot0), jnp.abs(dot1)),
                           jnp.abs(dot2))
        take = nms & (amax * jnp.abs(recip) <= 0.7)
        scale = jnp.where(take, -recip, 0.0)
        dx0 = dot0 * scale
        dx1 = dot1 * scale
        dx2 = dot2 * scale
        dy = 0.25 * ((bu1 * dx1 + bu2 * dx2) - k0 * dx0)

        y_ref[:, d] = (cd + dy) + jnp.where(nms, _BONUS, 0.0)
        coords_ref[:, 0, d] = float(d) + dx2
        coords_ref[:, 1, d] = widx + dx0
        coords_ref[:, 2, d] = hidx + dx1


def kernel(x):
    B, C, D, H, W = x.shape
    xs = x.reshape(B * C, D, H, W)
    HT = 64
    n_tiles = H // HT
    rpb = HT // 8
    coords, y = pl.pallas_call(
        functools.partial(_stencil_kernel, HT, n_tiles),
        grid=(n_tiles,),
        in_specs=[
            pl.BlockSpec((B * C, D, HT, W), lambda i: (0, 0, i, 0)),
            pl.BlockSpec((B * C, D, 8, W),
                         lambda i: (0, 0, jnp.maximum(i * rpb - 1, 0), 0)),
            pl.BlockSpec((B * C, D, 8, W),
                         lambda i: (0, 0, jnp.minimum((i + 1) * rpb, H // 8 - 1), 0)),
        ],
        out_specs=[
            pl.BlockSpec((B * C, 3, D, HT, W), lambda i: (0, 0, 0, i, 0)),
            pl.BlockSpec((B * C, D, HT, W), lambda i: (0, 0, i, 0)),
        ],
        out_shape=[
            jax.ShapeDtypeStruct((B * C, 3, D, H, W), x.dtype),
            jax.ShapeDtypeStruct((B * C, D, H, W), x.dtype),
        ],
    )(xs, xs, xs)
    return coords.reshape(B, C, 3, D, H, W), y.reshape(B, C, D, H, W)


# speedup vs baseline: 1.0795x; 1.0795x over previous
import functools

import jax
import jax.numpy as jnp
from jax.experimental import pallas as pl

_BONUS = 10.0


def _wp(v):
    return jnp.concatenate([v[..., 1:], v[..., -1:]], axis=-1)


def _wm(v):
    return jnp.concatenate([v[..., :1], v[..., :-1]], axis=-1)


def _stencil_kernel(ht, n_tiles, xc_ref, xt_ref, xb_ref, coords_ref, y_ref):
    i = pl.program_id(0)
    rs = i * ht
    c = xc_ref[...]
    trow = xt_ref[:, :, pl.ds(jnp.where(i == 0, 0, 7), 1), :]
    brow = xb_ref[:, :, pl.ds(jnp.where(i == n_tiles - 1, 7, 0), 1), :]

    CHp = jnp.concatenate([c[:, :, 1:], brow], axis=2)
    CHm = jnp.concatenate([trow, c[:, :, :-1]], axis=2)
    CWp = _wp(c)
    CWm = _wm(c)

    K = CWm - CWp
    L = CHp - CHm

    ktop = _wm(trow) - _wp(trow)
    kbot = _wm(brow) - _wp(brow)
    KHm = jnp.concatenate([ktop, K[:, :, :-1]], axis=2)
    KHp = jnp.concatenate([K[:, :, 1:], kbot], axis=2)

    m1 = jnp.maximum(jnp.maximum(CHm, c), CHp)
    mh = jnp.maximum(jnp.maximum(_wm(m1), m1), _wp(m1))

    sh = (c.shape[0], c.shape[2], c.shape[3])
    hidx = (rs + jax.lax.broadcasted_iota(jnp.int32, sh, 1)).astype(c.dtype)
    widx = jax.lax.broadcasted_iota(jnp.int32, sh, 2).astype(c.dtype)

    for d in range(4):
        dm = max(d - 1, 0)
        dp = min(d + 1, 3)
        cd = c[:, d]
        two_c = cd + cd
        m00 = 4.0 * ((CWm[:, d] + CWp[:, d]) - two_c)
        m11 = 4.0 * ((CHm[:, d] + CHp[:, d]) - two_c)
        m22 = 4.0 * ((c[:, dm] + c[:, dp]) - two_c)
        m01 = KHm[:, d] - KHp[:, d]
        m12 = L[:, dm] - L[:, dp]
        m02 = K[:, dp] - K[:, dm]
        k0 = K[:, d]
        bu1 = L[:, d]
        bu2 = c[:, dp] - c[:, dm]

        a00 = m11 * m22 - m12 * m12
        a01 = m02 * m12 - m01 * m22
        a02 = m01 * m12 - m02 * m11
        a11 = m00 * m22 - m02 * m02
        a12 = m01 * m02 - m00 * m12
        a22 = m00 * m11 - m01 * m01
        det = m00 * a00 + m01 * a01 + m02 * a02
        valid = det != 0.0
        recip = 2.0 / jnp.where(valid, det, 1.0)

        dot0 = (a01 * bu1 + a02 * bu2) - a00 * k0
        dot1 = (a11 * bu1 + a12 * bu2) - a01 * k0
        dot2 = (a12 * bu1 + a22 * bu2) - a02 * k0

        md = jnp.maximum(jnp.maximum(mh[:, dm], mh[:, d]), mh[:, dp])
        nms = (cd == md) & valid

        amax = jnp.maximum(jnp.maximum(jnp.abs(dot0), jnp.abs(dot1)),
                           jnp.abs(dot2))
        take = nms & (amax * jnp.abs(recip) <= 0.7)
        scale = jnp.where(take, -recip, 0.0)
        dx0 = dot0 * scale
        dx1 = dot1 * scale
        dx2 = dot2 * scale
        dy = 0.25 * ((bu1 * dx1 + bu2 * dx2) - k0 * dx0)

        y_ref[:, d] = (cd + dy) + jnp.where(nms, _BONUS, 0.0)
        coords_ref[:, 0, d] = float(d) + dx2
        coords_ref[:, 1, d] = widx + dx0
        coords_ref[:, 2, d] = hidx + dx1


def kernel(x):
    B, C, D, H, W = x.shape
    xs = x.reshape(B * C, D, H, W)
    HT = 64
    n_tiles = H // HT
    rpb = HT // 8
    coords, y = pl.pallas_call(
        functools.partial(_stencil_kernel, HT, n_tiles),
        grid=(n_tiles,),
        in_specs=[
            pl.BlockSpec((B * C, D, HT, W), lambda i: (0, 0, i, 0)),
            pl.BlockSpec((B * C, D, 8, W),
                         lambda i: (0, 0, jnp.maximum(i * rpb - 1, 0), 0)),
            pl.BlockSpec((B * C, D, 8, W),
                         lambda i: (0, 0, jnp.minimum((i + 1) * rpb, H // 8 - 1), 0)),
        ],
        out_specs=[
            pl.BlockSpec((B * C, 3, D, HT, W), lambda i: (0, 0, 0, i, 0)),
            pl.BlockSpec((B * C, D, HT, W), lambda i: (0, 0, i, 0)),
        ],
        out_shape=[
            jax.ShapeDtypeStruct((B * C, 3, D, H, W), x.dtype),
            jax.ShapeDtypeStruct((B * C, D, H, W), x.dtype),
        ],
    )(xs, xs, xs)
    return coords.reshape(B, C, 3, D, H, W), y.reshape(B, C, D, H, W)
